# baseline (speedup 1.0000x reference)
import functools

import jax
import jax.numpy as jnp
from jax import lax
from jax.experimental import pallas as pl
from jax.experimental.pallas import tpu as pltpu
from jax.experimental.pallas import tpu_sc as plsc

N = 10000
E = 640000
F = 64
CH = 128
NC = 2
NS = 16
NW = NC * NS
CHUNKS = E // CH
STR = 624
TAIL = N - NS * STR

_MESH = plsc.VectorSubcoreMesh(
    core_axis_name="c", subcore_axis_name="s", num_cores=NC, num_subcores=NS)

_SC_PARAMS = pltpu.CompilerParams(use_tc_tiling_on_sc=False)


def _zero_shared(zb_v, acc_sh, s, width):
    def zrow(i, _):
        for jj in range(width // 16):
            zb_v[i, pl.ds(jj * 16, 16)] = jnp.zeros((16,), jnp.float32)
        return 0
    lax.fori_loop(0, 48, zrow, 0)
    for i in range(STR // 48):
        pltpu.sync_copy(zb_v, acc_sh.at[pl.ds(s * STR + i * 48, 48)])

    @pl.when(s == NS - 1)
    def _():
        pltpu.sync_copy(zb_v.at[pl.ds(0, TAIL)], acc_sh.at[pl.ds(NS * STR, TAIL)])


def _copy_out(acc_sh, out_hbm, c, s):
    r0 = s * STR
    pltpu.sync_copy(acc_sh.at[pl.ds(r0, STR)], out_hbm.at[c, pl.ds(r0, STR)])

    @pl.when(s == NS - 1)
    def _():
        pltpu.sync_copy(acc_sh.at[pl.ds(NS * STR, TAIL)],
                        out_hbm.at[c, pl.ds(NS * STR, TAIL)])


SUB = 4
CPW = SUB * CH
NCH = E // CPW


def _edge_body(p_hbm, src_hbm, dst_hbm, attr_hbm, w1w_hbm, out_hbm,
               src_v, dst_v, attr_v, rows_v, w1w_v, zb_v, sem, acc_sh):
    c = lax.axis_index("c")
    s = lax.axis_index("s")
    wid = c * NS + s

    _zero_shared(zb_v, acc_sh, s, F)
    pltpu.sync_copy(w1w_hbm, w1w_v)
    plsc.subcore_barrier()

    wv = [[w1w_v[i, pl.ds(jj * 16, 16)] for i in range(3)] for jj in range(4)]

    nt = (NCH - wid + NW - 1) // NW

    def chunk(t, _):
        cr = (wid + NW * t) * SUB
        pltpu.sync_copy(src_hbm.at[pl.ds(cr, SUB)], src_v)
        pltpu.sync_copy(dst_hbm.at[pl.ds(cr, SUB)], dst_v)
        pltpu.sync_copy(attr_hbm.at[pl.ds(cr * CH * 16, CPW * 16)], attr_v)
        gd = [pltpu.async_copy(p_hbm.at[src_v.at[i]],
                               rows_v.at[pl.ds(i * CH, CH)], sem)
              for i in range(SUB)]
        for d in gd:
            d.wait()

        @plsc.parallel_loop(0, CPW, 1, unroll=4)
        def edge(k):
            av = attr_v[pl.ds(k * 16, 16)]
            w0 = av[0]
            w1 = av[1]
            w2 = av[2]
            for jj in range(4):
                sl = pl.ds(jj * 16, 16)
                r = rows_v[k, sl]
                r = r + w0 * wv[jj][0] + w1 * wv[jj][1] + w2 * wv[jj][2]
                rows_v[k, sl] = jnp.maximum(r, 0.01 * r)

        sd = [pltpu.async_copy(rows_v.at[pl.ds(i * CH, CH)],
                               acc_sh.at[dst_v.at[i]], sem, add=True)
              for i in range(SUB)]
        for d in sd:
            d.wait()
        return 0
    lax.fori_loop(0, nt, chunk, 0)

    plsc.subcore_barrier()
    _copy_out(acc_sh, out_hbm, c, s)


_edge_pass = pl.kernel(
    _edge_body,
    out_type=jax.ShapeDtypeStruct((NC, N, F), jnp.float32),
    mesh=_MESH,
    scratch_types=[
        pltpu.VMEM((SUB, CH), jnp.int32),
        pltpu.VMEM((SUB, CH), jnp.int32),
        pltpu.VMEM((CPW * 16,), jnp.float32),
        pltpu.VMEM((CPW, F), jnp.float32),
        pltpu.VMEM((3, F), jnp.float32),
        pltpu.VMEM((48, F), jnp.float32),
        pltpu.SemaphoreType.DMA,
        pltpu.VMEM_SHARED((N, F), jnp.float32),
    ],
    compiler_params=_SC_PARAMS,
)


def _deg_body(dst_hbm, out_hbm, dst_v, ones_v, zb_v, sem, acc_sh):
    c = lax.axis_index("c")
    s = lax.axis_index("s")
    wid = c * NS + s

    _zero_shared(zb_v, acc_sh, s, 16)

    def orow(i, _):
        ones_v[i, pl.ds(0, 16)] = jnp.ones((16,), jnp.float32)
        return 0
    lax.fori_loop(0, CH, orow, 0)
    plsc.subcore_barrier()

    nt = (NCH - wid + NW - 1) // NW

    def chunk(t, _):
        cr = (wid + NW * t) * SUB
        pltpu.sync_copy(dst_hbm.at[pl.ds(cr, SUB)], dst_v)
        sd = [pltpu.async_copy(ones_v, acc_sh.at[dst_v.at[i]], sem, add=True)
              for i in range(SUB)]
        for d in sd:
            d.wait()
        return 0
    lax.fori_loop(0, nt, chunk, 0)

    plsc.subcore_barrier()
    _copy_out(acc_sh, out_hbm, c, s)


_deg_pass = pl.kernel(
    _deg_body,
    out_type=jax.ShapeDtypeStruct((NC, N, 16), jnp.float32),
    mesh=_MESH,
    scratch_types=[
        pltpu.VMEM((SUB, CH), jnp.int32),
        pltpu.VMEM((CH, 16), jnp.float32),
        pltpu.VMEM((48, 16), jnp.float32),
        pltpu.SemaphoreType.DMA,
        pltpu.VMEM_SHARED((N, 16), jnp.float32),
    ],
    compiler_params=_SC_PARAMS,
)


def _split_kernel(ei_ref, s_ref, d_ref):
    s_ref[...] = ei_ref[0]
    d_ref[...] = ei_ref[1]


def _split_call(ei3):
    nr = E // CH
    rb = nr // 5
    return pl.pallas_call(
        _split_kernel,
        grid=(5,),
        in_specs=[pl.BlockSpec((2, rb, CH), lambda i: (0, i, 0))],
        out_specs=[
            pl.BlockSpec((rb, CH), lambda i: (i, 0)),
            pl.BlockSpec((rb, CH), lambda i: (i, 0)),
        ],
        out_shape=[
            jax.ShapeDtypeStruct((nr, CH), jnp.int32),
            jax.ShapeDtypeStruct((nr, CH), jnp.int32),
        ],
    )(ei3)

_RB = 1000
_GRID = N // _RB

def _embed_kernel(gate_ref, emb_ref, w1hT_ref, h_ref, p_ref):
    ids = gate_ref[0]
    iot = lax.broadcasted_iota(jnp.int32, (128, _RB), 0)
    ohT = (iot == ids).astype(jnp.float32)
    h = lax.dot_general(ohT, emb_ref[...], (((0,), (0,)), ((), ())),
                        preferred_element_type=jnp.float32)
    h_ref[...] = h
    p_ref[...] = jnp.dot(h, w1hT_ref[...], preferred_element_type=jnp.float32)


def _embed_call(gate3, emb, w1hT):
    return pl.pallas_call(
        _embed_kernel,
        grid=(_GRID,),
        in_specs=[
            pl.BlockSpec((1, 1, _RB), lambda i: (i, 0, 0)),
            pl.BlockSpec((128, 128), lambda i: (0, 0)),
            pl.BlockSpec((128, F), lambda i: (0, 0)),
        ],
        out_specs=[
            pl.BlockSpec((_RB, 128), lambda i: (i, 0)),
            pl.BlockSpec((_RB, F), lambda i: (i, 0)),
        ],
        out_shape=[
            jax.ShapeDtypeStruct((N, 128), jnp.float32),
            jax.ShapeDtypeStruct((N, F), jnp.float32),
        ],
    )(gate3, emb, w1hT)


def _layer_kernel(last, h_ref, a0_ref, a1_ref, d0_ref, d1_ref,
                  w2aT_ref, w2bT_ref, b2_ref, w1hTn_ref, ho_ref, po_ref):
    deg = d0_ref[:, 0:1] + d1_ref[:, 0:1]
    inv = 1.0 / jnp.maximum(deg, 1.0)
    hN = (a0_ref[...] + a1_ref[...]) * inv
    z = (jnp.dot(h_ref[...], w2aT_ref[...], preferred_element_type=jnp.float32)
         + jnp.dot(hN, w2bT_ref[...], preferred_element_type=jnp.float32)
         + b2_ref[...])
    if last:
        ho_ref[...] = z
        po_ref[...] = jnp.zeros_like(po_ref)
    else:
        hn = jnp.maximum(z, 0.0)
        ho_ref[...] = hn
        po_ref[...] = jnp.dot(hn, w1hTn_ref[...],
                              preferred_element_type=jnp.float32)


def _layer_call(h, a0, a1, d0, d1, w2aT, w2bT, b2, w1hTn, last):
    dout = w2aT.shape[1]
    return pl.pallas_call(
        functools.partial(_layer_kernel, last),
        grid=(_GRID,),
        in_specs=[
            pl.BlockSpec((_RB, 128), lambda i: (i, 0)),
            pl.BlockSpec((_RB, F), lambda i: (i, 0)),
            pl.BlockSpec((_RB, F), lambda i: (i, 0)),
            pl.BlockSpec((_RB, 16), lambda i: (i, 0)),
            pl.BlockSpec((_RB, 16), lambda i: (i, 0)),
            pl.BlockSpec((128, dout), lambda i: (0, 0)),
            pl.BlockSpec((F, dout), lambda i: (0, 0)),
            pl.BlockSpec((1, dout), lambda i: (0, 0)),
            pl.BlockSpec((dout, F), lambda i: (0, 0)),
        ],
        out_specs=[
            pl.BlockSpec((_RB, dout), lambda i: (i, 0)),
            pl.BlockSpec((_RB, F), lambda i: (i, 0)),
        ],
        out_shape=[
            jax.ShapeDtypeStruct((N, dout), jnp.float32),
            jax.ShapeDtypeStruct((N, F), jnp.float32),
        ],
    )(h, a0, a1, d0, d1, w2aT, w2bT, b2, w1hTn)


def kernel(gate_type, edge_index, edge_attr, emb,
           W1_1, W2_1, b2_1, W1_2, W2_2, b2_2, W1_3, W2_3, b2_3,
           W1_4, W2_4, b2_4, W1_5, W2_5, b2_5):
    W1s = [W1_1, W1_2, W1_3, W1_4, W1_5]
    W2s = [W2_1, W2_2, W2_3, W2_4, W2_5]
    b2s = [b2_1, b2_2, b2_3, b2_4, b2_5]

    src2, dst2 = _split_call(edge_index.reshape(2, E // CH, CH))
    attrf = jnp.pad(edge_attr, ((0, 0), (0, 13))).reshape(-1)
    gate3 = gate_type.reshape(_GRID, 1, _RB)

    w1hT = [w.T[:128] for w in W1s]
    w1w = [w.T[128:] for w in W1s]
    w2aT = [w.T[:128] for w in W2s]
    w2bT = [w.T[128:] for w in W2s]
    b2r = [b.reshape(1, -1) for b in b2s]

    degs = _deg_pass(dst2)
    d0, d1 = degs[0], degs[1]

    h, p = _embed_call(gate3, emb, w1hT[0])
    for l in range(5):
        accs = _edge_pass(p, src2, dst2, attrf, w1w[l])
        last = l == 4
        w1hTn = w1hT[l + 1] if not last else jnp.zeros((16, F), jnp.float32)
        h, p = _layer_call(h, accs[0], accs[1], d0, d1,
                           w2aT[l], w2bT[l], b2r[l], w1hTn, last)
    return h


# speedup vs baseline: 1.0011x; 1.0011x over previous
import functools

import jax
import jax.numpy as jnp
from jax import lax
from jax.experimental import pallas as pl
from jax.experimental.pallas import tpu as pltpu
from jax.experimental.pallas import tpu_sc as plsc

N = 10000
E = 640000
F = 64
CH = 128
NC = 2
NS = 16
NW = NC * NS
CHUNKS = E // CH
STR = 624
TAIL = N - NS * STR

_MESH = plsc.VectorSubcoreMesh(
    core_axis_name="c", subcore_axis_name="s", num_cores=NC, num_subcores=NS)

_SC_PARAMS = pltpu.CompilerParams(use_tc_tiling_on_sc=False)


def _zero_shared(zb_v, acc_sh, s, width):
    def zrow(i, _):
        for jj in range(width // 16):
            zb_v[i, pl.ds(jj * 16, 16)] = jnp.zeros((16,), jnp.float32)
        return 0
    lax.fori_loop(0, 48, zrow, 0)
    for i in range(STR // 48):
        pltpu.sync_copy(zb_v, acc_sh.at[pl.ds(s * STR + i * 48, 48)])

    @pl.when(s == NS - 1)
    def _():
        pltpu.sync_copy(zb_v.at[pl.ds(0, TAIL)], acc_sh.at[pl.ds(NS * STR, TAIL)])


def _copy_out(acc_sh, out_hbm, c, s):
    r0 = s * STR
    pltpu.sync_copy(acc_sh.at[pl.ds(r0, STR)], out_hbm.at[c, pl.ds(r0, STR)])

    @pl.when(s == NS - 1)
    def _():
        pltpu.sync_copy(acc_sh.at[pl.ds(NS * STR, TAIL)],
                        out_hbm.at[c, pl.ds(NS * STR, TAIL)])


SUB = 4
CPW = SUB * CH
NCH = E // CPW


def _edge_body(p_hbm, src_hbm, dst_hbm, attr_hbm, w1w_hbm, out_hbm,
               src_v, dst_v, attr_v, rows_v, w1w_v, zb_v, sem, acc_sh):
    c = lax.axis_index("c")
    s = lax.axis_index("s")
    wid = c * NS + s

    _zero_shared(zb_v, acc_sh, s, F)
    pltpu.sync_copy(w1w_hbm, w1w_v)
    plsc.subcore_barrier()

    wv = [[w1w_v[i, pl.ds(jj * 16, 16)] for i in range(3)] for jj in range(4)]

    nt = (NCH - wid + NW - 1) // NW

    def chunk(t, _):
        cr = (wid + NW * t) * SUB
        pltpu.sync_copy(src_hbm.at[pl.ds(cr, SUB)], src_v)
        pltpu.sync_copy(dst_hbm.at[pl.ds(cr, SUB)], dst_v)
        pltpu.sync_copy(attr_hbm.at[pl.ds(cr * CH * 16, CPW * 16)], attr_v)
        gd = [pltpu.async_copy(p_hbm.at[src_v.at[i]],
                               rows_v.at[pl.ds(i * CH, CH)], sem)
              for i in range(SUB)]
        for d in gd:
            d.wait()

        @plsc.parallel_loop(0, CPW, 1, unroll=4)
        def edge(k):
            av = attr_v[pl.ds(k * 16, 16)]
            w0 = av[0]
            w1 = av[1]
            w2 = av[2]
            for jj in range(4):
                sl = pl.ds(jj * 16, 16)
                r = rows_v[k, sl]
                r = r + w0 * wv[jj][0] + w1 * wv[jj][1] + w2 * wv[jj][2]
                rows_v[k, sl] = jnp.maximum(r, 0.01 * r)

        sd = [pltpu.async_copy(rows_v.at[pl.ds(i * CH, CH)],
                               acc_sh.at[dst_v.at[i]], sem, add=True)
              for i in range(SUB)]
        for d in sd:
            d.wait()
        return 0
    lax.fori_loop(0, nt, chunk, 0)

    plsc.subcore_barrier()
    _copy_out(acc_sh, out_hbm, c, s)


_edge_pass = pl.kernel(
    _edge_body,
    out_type=jax.ShapeDtypeStruct((NC, N, F), jnp.float32),
    mesh=_MESH,
    scratch_types=[
        pltpu.VMEM((SUB, CH), jnp.int32),
        pltpu.VMEM((SUB, CH), jnp.int32),
        pltpu.VMEM((CPW * 16,), jnp.float32),
        pltpu.VMEM((CPW, F), jnp.float32),
        pltpu.VMEM((3, F), jnp.float32),
        pltpu.VMEM((48, F), jnp.float32),
        pltpu.SemaphoreType.DMA,
        pltpu.VMEM_SHARED((N, F), jnp.float32),
    ],
    compiler_params=_SC_PARAMS,
)


def _deg_body(dst_hbm, out_hbm, dst_v, ones_v, zb_v, sem, acc_sh):
    c = lax.axis_index("c")
    s = lax.axis_index("s")
    wid = c * NS + s

    _zero_shared(zb_v, acc_sh, s, 16)

    def orow(i, _):
        ones_v[i, pl.ds(0, 16)] = jnp.ones((16,), jnp.float32)
        return 0
    lax.fori_loop(0, CH, orow, 0)
    plsc.subcore_barrier()

    nt = (NCH - wid + NW - 1) // NW

    def chunk(t, _):
        cr = (wid + NW * t) * SUB
        pltpu.sync_copy(dst_hbm.at[pl.ds(cr, SUB)], dst_v)
        sd = [pltpu.async_copy(ones_v, acc_sh.at[dst_v.at[i]], sem, add=True)
              for i in range(SUB)]
        for d in sd:
            d.wait()
        return 0
    lax.fori_loop(0, nt, chunk, 0)

    plsc.subcore_barrier()
    _copy_out(acc_sh, out_hbm, c, s)


_deg_pass = pl.kernel(
    _deg_body,
    out_type=jax.ShapeDtypeStruct((NC, N, 16), jnp.float32),
    mesh=_MESH,
    scratch_types=[
        pltpu.VMEM((SUB, CH), jnp.int32),
        pltpu.VMEM((CH, 16), jnp.float32),
        pltpu.VMEM((48, 16), jnp.float32),
        pltpu.SemaphoreType.DMA,
        pltpu.VMEM_SHARED((N, 16), jnp.float32),
    ],
    compiler_params=_SC_PARAMS,
)


def _split_kernel(ei_ref, s_ref, d_ref):
    s_ref[...] = ei_ref[0]
    d_ref[...] = ei_ref[1]


def _split_call(ei3):
    nr = E // CH
    rb = nr // 5
    return pl.pallas_call(
        _split_kernel,
        grid=(5,),
        in_specs=[pl.BlockSpec((2, rb, CH), lambda i: (0, i, 0))],
        out_specs=[
            pl.BlockSpec((rb, CH), lambda i: (i, 0)),
            pl.BlockSpec((rb, CH), lambda i: (i, 0)),
        ],
        out_shape=[
            jax.ShapeDtypeStruct((nr, CH), jnp.int32),
            jax.ShapeDtypeStruct((nr, CH), jnp.int32),
        ],
    )(ei3)

_RB = 1000
_GRID = N // _RB

def _embed_kernel(gate_ref, emb_ref, w1hT_ref, h_ref, p_ref):
    ids = gate_ref[0]
    iot = lax.broadcasted_iota(jnp.int32, (128, _RB), 0)
    ohT = (iot == ids).astype(jnp.float32)
    h = lax.dot_general(ohT, emb_ref[...], (((0,), (0,)), ((), ())),
                        preferred_element_type=jnp.float32)
    h_ref[...] = h
    p_ref[...] = jnp.dot(h, w1hT_ref[...], preferred_element_type=jnp.float32)


def _embed_call(gate3, emb, w1hT):
    return pl.pallas_call(
        _embed_kernel,
        grid=(_GRID,),
        in_specs=[
            pl.BlockSpec((1, 1, _RB), lambda i: (i, 0, 0)),
            pl.BlockSpec((128, 128), lambda i: (0, 0)),
            pl.BlockSpec((128, F), lambda i: (0, 0)),
        ],
        out_specs=[
            pl.BlockSpec((_RB, 128), lambda i: (i, 0)),
            pl.BlockSpec((_RB, F), lambda i: (i, 0)),
        ],
        out_shape=[
            jax.ShapeDtypeStruct((N, 128), jnp.float32),
            jax.ShapeDtypeStruct((N, F), jnp.float32),
        ],
    )(gate3, emb, w1hT)


def _layer_kernel(last, h_ref, a0_ref, a1_ref, d0_ref, d1_ref,
                  w2aT_ref, w2bT_ref, b2_ref, w1hTn_ref, ho_ref, po_ref):
    deg = d0_ref[:, 0:1] + d1_ref[:, 0:1]
    inv = 1.0 / jnp.maximum(deg, 1.0)
    hN = (a0_ref[...] + a1_ref[...]) * inv
    z = (jnp.dot(h_ref[...], w2aT_ref[...], preferred_element_type=jnp.float32)
         + jnp.dot(hN, w2bT_ref[...], preferred_element_type=jnp.float32)
         + b2_ref[...])
    if last:
        ho_ref[...] = z
        po_ref[...] = jnp.zeros_like(po_ref)
    else:
        hn = jnp.maximum(z, 0.0)
        ho_ref[...] = hn
        po_ref[...] = jnp.dot(hn, w1hTn_ref[...],
                              preferred_element_type=jnp.float32)


def _layer_call(h, a0, a1, d0, d1, w2aT, w2bT, b2, w1hTn, last):
    dout = w2aT.shape[1]
    return pl.pallas_call(
        functools.partial(_layer_kernel, last),
        grid=(_GRID,),
        in_specs=[
            pl.BlockSpec((_RB, 128), lambda i: (i, 0)),
            pl.BlockSpec((_RB, F), lambda i: (i, 0)),
            pl.BlockSpec((_RB, F), lambda i: (i, 0)),
            pl.BlockSpec((_RB, 16), lambda i: (i, 0)),
            pl.BlockSpec((_RB, 16), lambda i: (i, 0)),
            pl.BlockSpec((128, dout), lambda i: (0, 0)),
            pl.BlockSpec((F, dout), lambda i: (0, 0)),
            pl.BlockSpec((1, dout), lambda i: (0, 0)),
            pl.BlockSpec((dout, F), lambda i: (0, 0)),
        ],
        out_specs=[
            pl.BlockSpec((_RB, dout), lambda i: (i, 0)),
            pl.BlockSpec((_RB, F), lambda i: (i, 0)),
        ],
        out_shape=[
            jax.ShapeDtypeStruct((N, dout), jnp.float32),
            jax.ShapeDtypeStruct((N, F), jnp.float32),
        ],
    )(h, a0, a1, d0, d1, w2aT, w2bT, b2, w1hTn)


def kernel(gate_type, edge_index, edge_attr, emb,
           W1_1, W2_1, b2_1, W1_2, W2_2, b2_2, W1_3, W2_3, b2_3,
           W1_4, W2_4, b2_4, W1_5, W2_5, b2_5):
    W1s = [W1_1, W1_2, W1_3, W1_4, W1_5]
    W2s = [W2_1, W2_2, W2_3, W2_4, W2_5]
    b2s = [b2_1, b2_2, b2_3, b2_4, b2_5]

    ei = edge_index.reshape(2, E // CH, CH)
    src2 = jnp.maximum(ei[0], 0)
    dst2 = jnp.maximum(ei[1], 0)
    attrf = jnp.pad(edge_attr, ((0, 0), (0, 13))).reshape(-1)
    gate3 = gate_type.reshape(_GRID, 1, _RB)

    w1hT = [w.T[:128] for w in W1s]
    w1w = [w.T[128:] for w in W1s]
    w2aT = [w.T[:128] for w in W2s]
    w2bT = [w.T[128:] for w in W2s]
    b2r = [b.reshape(1, -1) for b in b2s]

    degs = _deg_pass(dst2)
    d0, d1 = degs[0], degs[1]

    h, p = _embed_call(gate3, emb, w1hT[0])
    for l in range(5):
        accs = _edge_pass(p, src2, dst2, attrf, w1w[l])
        last = l == 4
        w1hTn = w1hT[l + 1] if not last else jnp.zeros((16, F), jnp.float32)
        h, p = _layer_call(h, accs[0], accs[1], d0, d1,
                           w2aT[l], w2bT[l], b2r[l], w1hTn, last)
    return h


# speedup vs baseline: 3.1008x; 3.0975x over previous
import functools

import jax
import jax.numpy as jnp
from jax import lax
from jax.experimental import pallas as pl
from jax.experimental.pallas import tpu as pltpu
from jax.experimental.pallas import tpu_sc as plsc

N = 10000
E = 640000
F = 64
CH = 128
NC = 2
NS = 16
NW = NC * NS
CHUNKS = E // CH
STR = 624
TAIL = N - NS * STR

_MESH = plsc.VectorSubcoreMesh(
    core_axis_name="c", subcore_axis_name="s", num_cores=NC, num_subcores=NS)

_SC_PARAMS = pltpu.CompilerParams(use_tc_tiling_on_sc=False)


def _zero_shared(zb_v, acc_sh, s, width):
    def zrow(i, _):
        for jj in range(width // 16):
            zb_v[i, pl.ds(jj * 16, 16)] = jnp.zeros((16,), jnp.float32)
        return 0
    lax.fori_loop(0, 48, zrow, 0)
    for i in range(STR // 48):
        pltpu.sync_copy(zb_v, acc_sh.at[pl.ds(s * STR + i * 48, 48)])

    @pl.when(s == NS - 1)
    def _():
        pltpu.sync_copy(zb_v.at[pl.ds(0, TAIL)], acc_sh.at[pl.ds(NS * STR, TAIL)])


def _copy_out(acc_sh, out_hbm, c, s):
    r0 = s * STR
    pltpu.sync_copy(acc_sh.at[pl.ds(r0, STR)], out_hbm.at[c, pl.ds(r0, STR)])

    @pl.when(s == NS - 1)
    def _():
        pltpu.sync_copy(acc_sh.at[pl.ds(NS * STR, TAIL)],
                        out_hbm.at[c, pl.ds(NS * STR, TAIL)])


SUB = 4
CPW = SUB * CH
NCH = E // CPW


def _edge_body(p_hbm, src_hbm, dst_hbm, a0_hbm, a1_hbm, a2_hbm, w1w_hbm,
               out_hbm, src_v0, src_v1, dst_v0, dst_v1, attr_v0, attr_v1,
               rows_v0, rows_v1, w1w_v, zb_v, semi0, semi1, semg0, semg1,
               sems, acc_sh, deg_out_hbm=None, ones_v=None, zb16_v=None,
               deg_sh=None):
    c = lax.axis_index("c")
    s = lax.axis_index("s")
    wid = c * NS + s
    with_deg = deg_out_hbm is not None
    src_vs = (src_v0, src_v1)
    dst_vs = (dst_v0, dst_v1)
    attr_vs = (attr_v0, attr_v1)
    rows_vs = (rows_v0, rows_v1)
    semis = (semi0, semi1)
    semgs = (semg0, semg1)

    _zero_shared(zb_v, acc_sh, s, F)
    if with_deg:
        _zero_shared(zb16_v, deg_sh, s, 16)

        def orow(i, _):
            ones_v[i, pl.ds(0, 16)] = jnp.ones((16,), jnp.float32)
            return 0
        lax.fori_loop(0, CH, orow, 0)
    pltpu.sync_copy(w1w_hbm, w1w_v)
    plsc.subcore_barrier()

    wv = [[w1w_v[i, pl.ds(jj * 16, 16)] for i in range(3)] for jj in range(4)]

    nt = (NCH - wid + NW - 1) // NW

    def _load_list(t, b, mk):
        cr = (wid + NW * t) * SUB
        base = cr * CH
        d = [mk(src_hbm.at[pl.ds(cr, SUB)], src_vs[b], semis[b]),
             mk(dst_hbm.at[pl.ds(cr, SUB)], dst_vs[b], semis[b])]
        d += [mk(a.at[pl.ds(base, CPW)],
                 attr_vs[b].at[i, pl.ds(0, CPW)], semis[b])
              for i, a in enumerate((a0_hbm, a1_hbm, a2_hbm))]
        return d

    def _gather_list(b, mk):
        return [mk(p_hbm.at[src_vs[b].at[i]],
                   rows_vs[b].at[pl.ds(i * CH, CH)], semgs[b])
                for i in range(SUB)]

    for d in _load_list(0, 0, pltpu.async_copy):
        d.wait()
    _gather_list(0, pltpu.async_copy)
    _load_list(1, 1, pltpu.async_copy)

    def pair(i, _):
        for b in (0, 1):
            t = 2 * i + b
            b1 = 1 - b

            @pl.when(t < nt)
            def _():
                for d in _gather_list(b, pltpu.make_async_copy):
                    d.wait()

                @pl.when(t + 1 < nt)
                def _():
                    for d in _load_list(t + 1, b1, pltpu.make_async_copy):
                        d.wait()
                    _gather_list(b1, pltpu.async_copy)

                @plsc.parallel_loop(0, CPW, 1, unroll=4)
                def edge(k):
                    w0 = attr_vs[b][0, pl.ds(k, 16)][0]
                    w1 = attr_vs[b][1, pl.ds(k, 16)][0]
                    w2 = attr_vs[b][2, pl.ds(k, 16)][0]
                    for jj in range(4):
                        sl = pl.ds(jj * 16, 16)
                        r = rows_vs[b][k, sl]
                        r = r + w0 * wv[jj][0] + w1 * wv[jj][1] + w2 * wv[jj][2]
                        rows_vs[b][k, sl] = jnp.maximum(r, 0.01 * r)

                sd = [pltpu.async_copy(rows_vs[b].at[pl.ds(i * CH, CH)],
                                       acc_sh.at[dst_vs[b].at[i]], sems,
                                       add=True)
                      for i in range(SUB)]
                if with_deg:
                    sd += [pltpu.async_copy(ones_v,
                                            deg_sh.at[dst_vs[b].at[i]], sems,
                                            add=True)
                           for i in range(SUB)]
                for d in sd:
                    d.wait()

                @pl.when(t + 2 < nt)
                def _():
                    _load_list(t + 2, b, pltpu.async_copy)
        return 0
    lax.fori_loop(0, (nt + 1) // 2, pair, 0)

    plsc.subcore_barrier()
    _copy_out(acc_sh, out_hbm, c, s)
    if with_deg:
        _copy_out(deg_sh, deg_out_hbm, c, s)


_COMMON_SCRATCH = [
    pltpu.VMEM((SUB, CH), jnp.int32),
    pltpu.VMEM((SUB, CH), jnp.int32),
    pltpu.VMEM((SUB, CH), jnp.int32),
    pltpu.VMEM((SUB, CH), jnp.int32),
    pltpu.VMEM((3, CPW + 16), jnp.float32),
    pltpu.VMEM((3, CPW + 16), jnp.float32),
    pltpu.VMEM((CPW, F), jnp.float32),
    pltpu.VMEM((CPW, F), jnp.float32),
    pltpu.VMEM((3, F), jnp.float32),
    pltpu.VMEM((48, F), jnp.float32),
    pltpu.SemaphoreType.DMA,
    pltpu.SemaphoreType.DMA,
    pltpu.SemaphoreType.DMA,
    pltpu.SemaphoreType.DMA,
    pltpu.SemaphoreType.DMA,
    pltpu.VMEM_SHARED((N, F), jnp.float32),
]

_edge_pass = pl.kernel(
    _edge_body,
    out_type=jax.ShapeDtypeStruct((NC, N, F), jnp.float32),
    mesh=_MESH,
    scratch_types=list(_COMMON_SCRATCH),
    compiler_params=_SC_PARAMS,
)

def _edge_deg_body(p, src, dst, a0, a1, a2, w1w, out, deg_out, *scratch):
    common = scratch[:16]
    ones_v, zb16_v, deg_sh = scratch[16:]
    _edge_body(p, src, dst, a0, a1, a2, w1w, out, *common,
               deg_out_hbm=deg_out, ones_v=ones_v, zb16_v=zb16_v,
               deg_sh=deg_sh)


_edge_deg_pass = pl.kernel(
    _edge_deg_body,
    out_type=[jax.ShapeDtypeStruct((NC, N, F), jnp.float32),
              jax.ShapeDtypeStruct((NC, N, 16), jnp.float32)],
    mesh=_MESH,
    scratch_types=list(_COMMON_SCRATCH) + [
        pltpu.VMEM((CH, 16), jnp.float32),
        pltpu.VMEM((48, 16), jnp.float32),
        pltpu.VMEM_SHARED((N, 16), jnp.float32),
    ],
    compiler_params=_SC_PARAMS,
)


def _split_kernel(ei_ref, s_ref, d_ref):
    s_ref[...] = ei_ref[0]
    d_ref[...] = ei_ref[1]


def _split_call(ei3):
    nr = E // CH
    rb = nr // 5
    return pl.pallas_call(
        _split_kernel,
        grid=(5,),
        in_specs=[pl.BlockSpec((2, rb, CH), lambda i: (0, i, 0))],
        out_specs=[
            pl.BlockSpec((rb, CH), lambda i: (i, 0)),
            pl.BlockSpec((rb, CH), lambda i: (i, 0)),
        ],
        out_shape=[
            jax.ShapeDtypeStruct((nr, CH), jnp.int32),
            jax.ShapeDtypeStruct((nr, CH), jnp.int32),
        ],
    )(ei3)

_RB = 1000
_GRID = N // _RB

def _embed_kernel(gate_ref, emb_ref, w1hT_ref, h_ref, p_ref):
    ids = gate_ref[0]
    iot = lax.broadcasted_iota(jnp.int32, (128, _RB), 0)
    ohT = (iot == ids).astype(jnp.float32)
    h = lax.dot_general(ohT, emb_ref[...], (((0,), (0,)), ((), ())),
                        preferred_element_type=jnp.float32)
    h_ref[...] = h
    p_ref[...] = jnp.dot(h, w1hT_ref[...], preferred_element_type=jnp.float32)


def _embed_call(gate3, emb, w1hT):
    return pl.pallas_call(
        _embed_kernel,
        grid=(_GRID,),
        in_specs=[
            pl.BlockSpec((1, 1, _RB), lambda i: (i, 0, 0)),
            pl.BlockSpec((128, 128), lambda i: (0, 0)),
            pl.BlockSpec((128, F), lambda i: (0, 0)),
        ],
        out_specs=[
            pl.BlockSpec((_RB, 128), lambda i: (i, 0)),
            pl.BlockSpec((_RB, F), lambda i: (i, 0)),
        ],
        out_shape=[
            jax.ShapeDtypeStruct((N, 128), jnp.float32),
            jax.ShapeDtypeStruct((N, F), jnp.float32),
        ],
    )(gate3, emb, w1hT)


def _layer_kernel(last, h_ref, a0_ref, a1_ref, d0_ref, d1_ref,
                  w2aT_ref, w2bT_ref, b2_ref, w1hTn_ref, ho_ref, po_ref):
    deg = d0_ref[:, 0:1] + d1_ref[:, 0:1]
    inv = 1.0 / jnp.maximum(deg, 1.0)
    hN = (a0_ref[...] + a1_ref[...]) * inv
    z = (jnp.dot(h_ref[...], w2aT_ref[...], preferred_element_type=jnp.float32)
         + jnp.dot(hN, w2bT_ref[...], preferred_element_type=jnp.float32)
         + b2_ref[...])
    if last:
        ho_ref[...] = z
        po_ref[...] = jnp.zeros_like(po_ref)
    else:
        hn = jnp.maximum(z, 0.0)
        ho_ref[...] = hn
        po_ref[...] = jnp.dot(hn, w1hTn_ref[...],
                              preferred_element_type=jnp.float32)


def _layer_call(h, a0, a1, d0, d1, w2aT, w2bT, b2, w1hTn, last):
    dout = w2aT.shape[1]
    return pl.pallas_call(
        functools.partial(_layer_kernel, last),
        grid=(_GRID,),
        in_specs=[
            pl.BlockSpec((_RB, 128), lambda i: (i, 0)),
            pl.BlockSpec((_RB, F), lambda i: (i, 0)),
            pl.BlockSpec((_RB, F), lambda i: (i, 0)),
            pl.BlockSpec((_RB, 16), lambda i: (i, 0)),
            pl.BlockSpec((_RB, 16), lambda i: (i, 0)),
            pl.BlockSpec((128, dout), lambda i: (0, 0)),
            pl.BlockSpec((F, dout), lambda i: (0, 0)),
            pl.BlockSpec((1, dout), lambda i: (0, 0)),
            pl.BlockSpec((dout, F), lambda i: (0, 0)),
        ],
        out_specs=[
            pl.BlockSpec((_RB, dout), lambda i: (i, 0)),
            pl.BlockSpec((_RB, F), lambda i: (i, 0)),
        ],
        out_shape=[
            jax.ShapeDtypeStruct((N, dout), jnp.float32),
            jax.ShapeDtypeStruct((N, F), jnp.float32),
        ],
    )(h, a0, a1, d0, d1, w2aT, w2bT, b2, w1hTn)


def kernel(gate_type, edge_index, edge_attr, emb,
           W1_1, W2_1, b2_1, W1_2, W2_2, b2_2, W1_3, W2_3, b2_3,
           W1_4, W2_4, b2_4, W1_5, W2_5, b2_5):
    W1s = [W1_1, W1_2, W1_3, W1_4, W1_5]
    W2s = [W2_1, W2_2, W2_3, W2_4, W2_5]
    b2s = [b2_1, b2_2, b2_3, b2_4, b2_5]

    ei = edge_index.reshape(2, E // CH, CH)
    src2 = jnp.maximum(ei[0], 0)
    dst2 = jnp.maximum(ei[1], 0)
    a0 = jnp.maximum(edge_attr[:, 0], 0.0)
    a1 = jnp.maximum(edge_attr[:, 1], 0.0)
    a2 = jnp.maximum(edge_attr[:, 2], 0.0)
    gate3 = gate_type.reshape(_GRID, 1, _RB)

    w1hT = [w.T[:128] for w in W1s]
    w1w = [w.T[128:] for w in W1s]
    w2aT = [w.T[:128] for w in W2s]
    w2bT = [w.T[128:] for w in W2s]
    b2r = [b.reshape(1, -1) for b in b2s]

    h, p = _embed_call(gate3, emb, w1hT[0])
    for l in range(5):
        if l == 0:
            accs, degs = _edge_deg_pass(p, src2, dst2, a0, a1, a2, w1w[l])
            d0, d1 = degs[0], degs[1]
        else:
            accs = _edge_pass(p, src2, dst2, a0, a1, a2, w1w[l])
        last = l == 4
        w1hTn = w1hT[l + 1] if not last else jnp.zeros((16, F), jnp.float32)
        h, p = _layer_call(h, accs[0], accs[1], d0, d1,
                           w2aT[l], w2bT[l], b2r[l], w1hTn, last)
    return h
